# baseline (speedup 1.0000x reference)
import jax
import jax.numpy as jnp
from jax import lax
from jax.experimental import pallas as pl
from jax.experimental.pallas import tpu as pltpu

_QB = 512
_LOG2E = 1.4426950408889634


def kernel(x, w_theta, w_phi, w_g, w_o, gamma):
    B, C, H, W = x.shape
    N = H * W
    Hp, Wp = H // 2, W // 2
    M = Hp * Wp
    C8 = w_theta.shape[0]
    C2 = w_g.shape[0]
    CP = C8 + C2
    KG = C + C2 + 8
    QB = _QB

    x_flat = x.reshape(B, C, N)
    w_pg = jnp.concatenate([w_phi, w_g], axis=0)
    w_theta_s = (w_theta * _LOG2E).astype(jnp.float32)
    gamma_arr = jnp.reshape(gamma, (1,)).astype(jnp.float32)

    def prep_kernel(x_ref, w_pg_ref, w_th_ref, kg_ref, full_scr, pool_scr):
        xb = x_ref[0]
        full_scr[...] = jnp.dot(
            w_pg_ref[...], xb, preferred_element_type=jnp.float32
        )
        r = lax.broadcasted_iota(jnp.int32, (W, Wp), 0)
        c = lax.broadcasted_iota(jnp.int32, (W, Wp), 1)
        E0 = jnp.where(r == 2 * c, 1.0, 0.0).astype(jnp.float32)
        E1 = jnp.where(r == 2 * c + 1, 1.0, 0.0).astype(jnp.float32)
        for hp in range(Hp):
            a = full_scr[:, (2 * hp) * W:(2 * hp) * W + W]
            b = full_scr[:, (2 * hp + 1) * W:(2 * hp + 1) * W + W]
            hm = jnp.maximum(a, b)
            pooled = jnp.maximum(
                jnp.dot(hm, E0, preferred_element_type=jnp.float32),
                jnp.dot(hm, E1, preferred_element_type=jnp.float32),
            )
            pool_scr[:, hp * Wp:(hp + 1) * Wp] = pooled
        k_eff = lax.dot_general(
            w_th_ref[...], pool_scr[:C8, :],
            (((0,), (0,)), ((), ())),
            preferred_element_type=jnp.float32,
        )
        kg_ref[0, :C, :] = k_eff.astype(jnp.bfloat16)
        kg_ref[0, C:C + C2, :] = pool_scr[C8:, :].astype(jnp.bfloat16)
        rr = lax.broadcasted_iota(jnp.int32, (8, M), 0)
        kg_ref[0, C + C2:, :] = jnp.where(rr == 0, 1.0, 0.0).astype(jnp.bfloat16)

    kg = pl.pallas_call(
        prep_kernel,
        grid=(B,),
        in_specs=[
            pl.BlockSpec((1, C, N), lambda b: (b, 0, 0)),
            pl.BlockSpec((CP, C), lambda b: (0, 0)),
            pl.BlockSpec((C8, C), lambda b: (0, 0)),
        ],
        out_specs=pl.BlockSpec((1, KG, M), lambda b: (b, 0, 0)),
        out_shape=jax.ShapeDtypeStruct((B, KG, M), jnp.bfloat16),
        scratch_shapes=[
            pltpu.VMEM((CP, N), jnp.float32),
            pltpu.VMEM((CP, M), jnp.float32),
        ],
        compiler_params=pltpu.CompilerParams(
            dimension_semantics=("parallel",),
            vmem_limit_bytes=40 * 1024 * 1024,
        ),
        name="nl_attn_prep",
    )(x_flat, w_pg, w_theta_s)

    NC = 1
    QC = QB // NC

    def attn_kernel(x_ref, kg_ref, w_o_ref, gamma_ref, o_ref):
        xb = x_ref[0]
        k_eff = kg_ref[0, :C, :]
        g_aug = kg_ref[0, C:, :]
        xb_bf = xb.astype(jnp.bfloat16)

        def chunk(c):
            xc = xb_bf[:, c * QC:(c + 1) * QC]
            scores = lax.dot_general(
                xc, k_eff, (((0,), (0,)), ((), ())),
                preferred_element_type=jnp.float32,
            )
            scores_bf = scores.astype(jnp.bfloat16)
            mx = jnp.max(scores_bf, axis=-1, keepdims=True)
            e = jnp.exp2(scores_bf - mx)
            o_aug = lax.dot_general(
                g_aug, e, (((1,), (1,)), ((), ())),
                preferred_element_type=jnp.float32,
            )
            o_midT = o_aug[:C2, :]
            s_row = o_aug[C2:C2 + 1, :]
            oT = jnp.dot(
                w_o_ref[...], o_midT, preferred_element_type=jnp.float32)
            scale = gamma_ref[0] / s_row
            o_ref[0, :, c * QC:(c + 1) * QC] = (
                oT * scale + xb[:, c * QC:(c + 1) * QC])

        for c in range(NC):
            chunk(c)

    out_flat = pl.pallas_call(
        attn_kernel,
        grid=(B, N // QB),
        in_specs=[
            pl.BlockSpec((1, C, QB), lambda b, q: (b, 0, q)),
            pl.BlockSpec((1, KG, M), lambda b, q: (b, 0, 0)),
            pl.BlockSpec((C, C2), lambda b, q: (0, 0)),
            pl.BlockSpec(memory_space=pltpu.SMEM),
        ],
        out_specs=pl.BlockSpec((1, C, QB), lambda b, q: (b, 0, q)),
        out_shape=jax.ShapeDtypeStruct((B, C, N), jnp.float32),
        compiler_params=pltpu.CompilerParams(
            dimension_semantics=("parallel", "arbitrary"),
            vmem_limit_bytes=48 * 1024 * 1024,
        ),
        name="nl_attn_main",
    )(x_flat, kg, w_o, gamma_arr)

    return out_flat.reshape(B, C, H, W)


# speedup vs baseline: 1.1015x; 1.1015x over previous
import jax
import jax.numpy as jnp
from jax import lax
from jax.experimental import pallas as pl
from jax.experimental.pallas import tpu as pltpu

_QB = 1024
_LOG2E = 1.4426950408889634


def kernel(x, w_theta, w_phi, w_g, w_o, gamma):
    B, C, H, W = x.shape
    N = H * W
    Hp, Wp = H // 2, W // 2
    M = Hp * Wp
    C8 = w_theta.shape[0]
    C2 = w_g.shape[0]
    CP = C8 + C2
    KG = C + C2 + 8
    QB = _QB

    x_flat = x.reshape(B, C, N)
    w_pg = jnp.concatenate([w_phi, w_g], axis=0)
    w_theta_s = (w_theta * _LOG2E).astype(jnp.float32)
    gamma_arr = jnp.reshape(gamma, (1,)).astype(jnp.float32)

    def prep_kernel(x_ref, w_pg_ref, w_th_ref, kg_ref, full_scr, pool_scr):
        xb = x_ref[0]
        full_scr[...] = jnp.dot(
            w_pg_ref[...], xb, preferred_element_type=jnp.float32
        )
        r = lax.broadcasted_iota(jnp.int32, (W, Wp), 0)
        c = lax.broadcasted_iota(jnp.int32, (W, Wp), 1)
        E0 = jnp.where(r == 2 * c, 1.0, 0.0).astype(jnp.float32)
        E1 = jnp.where(r == 2 * c + 1, 1.0, 0.0).astype(jnp.float32)
        for hp in range(Hp):
            a = full_scr[:, (2 * hp) * W:(2 * hp) * W + W]
            b = full_scr[:, (2 * hp + 1) * W:(2 * hp + 1) * W + W]
            hm = jnp.maximum(a, b)
            pooled = jnp.maximum(
                jnp.dot(hm, E0, preferred_element_type=jnp.float32),
                jnp.dot(hm, E1, preferred_element_type=jnp.float32),
            )
            pool_scr[:, hp * Wp:(hp + 1) * Wp] = pooled
        k_eff = lax.dot_general(
            w_th_ref[...], pool_scr[:C8, :],
            (((0,), (0,)), ((), ())),
            preferred_element_type=jnp.float32,
        )
        kg_ref[0, :C, :] = k_eff.astype(jnp.bfloat16)
        kg_ref[0, C:C + C2, :] = pool_scr[C8:, :].astype(jnp.bfloat16)
        rr = lax.broadcasted_iota(jnp.int32, (8, M), 0)
        kg_ref[0, C + C2:, :] = jnp.where(rr == 0, 1.0, 0.0).astype(jnp.bfloat16)

    kg = pl.pallas_call(
        prep_kernel,
        grid=(B,),
        in_specs=[
            pl.BlockSpec((1, C, N), lambda b: (b, 0, 0)),
            pl.BlockSpec((CP, C), lambda b: (0, 0)),
            pl.BlockSpec((C8, C), lambda b: (0, 0)),
        ],
        out_specs=pl.BlockSpec((1, KG, M), lambda b: (b, 0, 0)),
        out_shape=jax.ShapeDtypeStruct((B, KG, M), jnp.bfloat16),
        scratch_shapes=[
            pltpu.VMEM((CP, N), jnp.float32),
            pltpu.VMEM((CP, M), jnp.float32),
        ],
        compiler_params=pltpu.CompilerParams(
            dimension_semantics=("parallel",),
            vmem_limit_bytes=40 * 1024 * 1024,
        ),
        name="nl_attn_prep",
    )(x_flat, w_pg, w_theta_s)

    NC = 1
    QC = QB // NC

    def attn_kernel(x_ref, kg_ref, w_o_ref, gamma_ref, o_ref):
        xb = x_ref[0]
        k_eff = kg_ref[0, :C, :]
        g_aug = kg_ref[0, C:, :]
        xb_bf = xb.astype(jnp.bfloat16)

        def chunk(c):
            xc = xb_bf[:, c * QC:(c + 1) * QC]
            scores = lax.dot_general(
                xc, k_eff, (((0,), (0,)), ((), ())),
                preferred_element_type=jnp.float32,
            )
            scores_bf = scores.astype(jnp.bfloat16)
            mx = jnp.max(scores_bf, axis=-1, keepdims=True)
            e = jnp.exp2(scores_bf - mx)
            o_aug = lax.dot_general(
                g_aug, e, (((1,), (1,)), ((), ())),
                preferred_element_type=jnp.float32,
            )
            o_midT = o_aug[:C2, :]
            s_row = o_aug[C2:C2 + 1, :]
            oT = jnp.dot(
                w_o_ref[...], o_midT, preferred_element_type=jnp.float32)
            scale = gamma_ref[0] / s_row
            o_ref[0, :, c * QC:(c + 1) * QC] = (
                oT * scale + xb[:, c * QC:(c + 1) * QC])

        for c in range(NC):
            chunk(c)

    out_flat = pl.pallas_call(
        attn_kernel,
        grid=(B, N // QB),
        in_specs=[
            pl.BlockSpec((1, C, QB), lambda b, q: (b, 0, q)),
            pl.BlockSpec((1, KG, M), lambda b, q: (b, 0, 0)),
            pl.BlockSpec((C, C2), lambda b, q: (0, 0)),
            pl.BlockSpec(memory_space=pltpu.SMEM),
        ],
        out_specs=pl.BlockSpec((1, C, QB), lambda b, q: (b, 0, q)),
        out_shape=jax.ShapeDtypeStruct((B, C, N), jnp.float32),
        compiler_params=pltpu.CompilerParams(
            dimension_semantics=("parallel", "arbitrary"),
            vmem_limit_bytes=48 * 1024 * 1024,
        ),
        name="nl_attn_main",
    )(x_flat, kg, w_o, gamma_arr)

    return out_flat.reshape(B, C, H, W)


# speedup vs baseline: 1.1312x; 1.0269x over previous
import jax
import jax.numpy as jnp
from jax import lax
from jax.experimental import pallas as pl
from jax.experimental.pallas import tpu as pltpu

_QB = 2048
_LOG2E = 1.4426950408889634


def kernel(x, w_theta, w_phi, w_g, w_o, gamma):
    B, C, H, W = x.shape
    N = H * W
    Hp, Wp = H // 2, W // 2
    M = Hp * Wp
    C8 = w_theta.shape[0]
    C2 = w_g.shape[0]
    CP = C8 + C2
    KG = C + C2 + 8
    QB = _QB

    x_flat = x.reshape(B, C, N)
    w_pg = jnp.concatenate([w_phi, w_g], axis=0)
    w_theta_s = (w_theta * _LOG2E).astype(jnp.float32)
    gamma_arr = jnp.reshape(gamma, (1,)).astype(jnp.float32)

    def prep_kernel(x_ref, w_pg_ref, w_th_ref, kg_ref, full_scr, pool_scr):
        xb = x_ref[0]
        full_scr[...] = jnp.dot(
            w_pg_ref[...], xb, preferred_element_type=jnp.float32
        )
        r = lax.broadcasted_iota(jnp.int32, (W, Wp), 0)
        c = lax.broadcasted_iota(jnp.int32, (W, Wp), 1)
        E0 = jnp.where(r == 2 * c, 1.0, 0.0).astype(jnp.float32)
        E1 = jnp.where(r == 2 * c + 1, 1.0, 0.0).astype(jnp.float32)
        for hp in range(Hp):
            a = full_scr[:, (2 * hp) * W:(2 * hp) * W + W]
            b = full_scr[:, (2 * hp + 1) * W:(2 * hp + 1) * W + W]
            hm = jnp.maximum(a, b)
            pooled = jnp.maximum(
                jnp.dot(hm, E0, preferred_element_type=jnp.float32),
                jnp.dot(hm, E1, preferred_element_type=jnp.float32),
            )
            pool_scr[:, hp * Wp:(hp + 1) * Wp] = pooled
        k_eff = lax.dot_general(
            w_th_ref[...], pool_scr[:C8, :],
            (((0,), (0,)), ((), ())),
            preferred_element_type=jnp.float32,
        )
        kg_ref[0, :C, :] = k_eff.astype(jnp.bfloat16)
        kg_ref[0, C:C + C2, :] = pool_scr[C8:, :].astype(jnp.bfloat16)
        rr = lax.broadcasted_iota(jnp.int32, (8, M), 0)
        kg_ref[0, C + C2:, :] = jnp.where(rr == 0, 1.0, 0.0).astype(jnp.bfloat16)

    kg = pl.pallas_call(
        prep_kernel,
        grid=(B,),
        in_specs=[
            pl.BlockSpec((1, C, N), lambda b: (b, 0, 0)),
            pl.BlockSpec((CP, C), lambda b: (0, 0)),
            pl.BlockSpec((C8, C), lambda b: (0, 0)),
        ],
        out_specs=pl.BlockSpec((1, KG, M), lambda b: (b, 0, 0)),
        out_shape=jax.ShapeDtypeStruct((B, KG, M), jnp.bfloat16),
        scratch_shapes=[
            pltpu.VMEM((CP, N), jnp.float32),
            pltpu.VMEM((CP, M), jnp.float32),
        ],
        compiler_params=pltpu.CompilerParams(
            dimension_semantics=("parallel",),
            vmem_limit_bytes=40 * 1024 * 1024,
        ),
        name="nl_attn_prep",
    )(x_flat, w_pg, w_theta_s)

    NC = 1
    QC = QB // NC

    def attn_kernel(x_ref, kg_ref, w_o_ref, gamma_ref, o_ref):
        xb = x_ref[0]
        k_eff = kg_ref[0, :C, :]
        g_aug = kg_ref[0, C:, :]
        xb_bf = xb.astype(jnp.bfloat16)

        def chunk(c):
            xc = xb_bf[:, c * QC:(c + 1) * QC]
            scores = lax.dot_general(
                xc, k_eff, (((0,), (0,)), ((), ())),
                preferred_element_type=jnp.float32,
            )
            scores_bf = scores.astype(jnp.bfloat16)
            mx = jnp.max(scores_bf, axis=-1, keepdims=True)
            e = jnp.exp2(scores_bf - mx)
            o_aug = lax.dot_general(
                g_aug, e, (((1,), (1,)), ((), ())),
                preferred_element_type=jnp.float32,
            )
            o_midT = o_aug[:C2, :]
            s_row = o_aug[C2:C2 + 1, :]
            oT = jnp.dot(
                w_o_ref[...], o_midT, preferred_element_type=jnp.float32)
            scale = gamma_ref[0] / s_row
            o_ref[0, :, c * QC:(c + 1) * QC] = (
                oT * scale + xb[:, c * QC:(c + 1) * QC])

        for c in range(NC):
            chunk(c)

    out_flat = pl.pallas_call(
        attn_kernel,
        grid=(B, N // QB),
        in_specs=[
            pl.BlockSpec((1, C, QB), lambda b, q: (b, 0, q)),
            pl.BlockSpec((1, KG, M), lambda b, q: (b, 0, 0)),
            pl.BlockSpec((C, C2), lambda b, q: (0, 0)),
            pl.BlockSpec(memory_space=pltpu.SMEM),
        ],
        out_specs=pl.BlockSpec((1, C, QB), lambda b, q: (b, 0, q)),
        out_shape=jax.ShapeDtypeStruct((B, C, N), jnp.float32),
        compiler_params=pltpu.CompilerParams(
            dimension_semantics=("parallel", "arbitrary"),
            vmem_limit_bytes=56 * 1024 * 1024,
        ),
        name="nl_attn_main",
    )(x_flat, kg, w_o, gamma_arr)

    return out_flat.reshape(B, C, H, W)


# speedup vs baseline: 1.1376x; 1.0057x over previous
import jax
import jax.numpy as jnp
from jax import lax
from jax.experimental import pallas as pl
from jax.experimental.pallas import tpu as pltpu

_QB = 2048
_LOG2E = 1.4426950408889634


def kernel(x, w_theta, w_phi, w_g, w_o, gamma):
    B, C, H, W = x.shape
    N = H * W
    Hp, Wp = H // 2, W // 2
    M = Hp * Wp
    C8 = w_theta.shape[0]
    C2 = w_g.shape[0]
    CP = C8 + C2
    PAD = 16
    KG = C + PAD + C2 + PAD
    QB = _QB

    x_flat = x.reshape(B, C, N)
    w_pg = jnp.concatenate([w_phi, w_g], axis=0)
    w_theta_s = (w_theta * _LOG2E).astype(jnp.float32)
    gamma_arr = jnp.reshape(gamma, (1,)).astype(jnp.float32)

    def prep_kernel(x_ref, w_pg_ref, w_th_ref, kg_ref, km_ref,
                    full_scr, pool_scr):
        xb = x_ref[0]
        full_scr[...] = jnp.dot(
            w_pg_ref[...], xb, preferred_element_type=jnp.float32
        )
        r = lax.broadcasted_iota(jnp.int32, (W, Wp), 0)
        c = lax.broadcasted_iota(jnp.int32, (W, Wp), 1)
        E0 = jnp.where(r == 2 * c, 1.0, 0.0).astype(jnp.float32)
        E1 = jnp.where(r == 2 * c + 1, 1.0, 0.0).astype(jnp.float32)
        for hp in range(Hp):
            a = full_scr[:, (2 * hp) * W:(2 * hp) * W + W]
            b = full_scr[:, (2 * hp + 1) * W:(2 * hp + 1) * W + W]
            hm = jnp.maximum(a, b)
            pooled = jnp.maximum(
                jnp.dot(hm, E0, preferred_element_type=jnp.float32),
                jnp.dot(hm, E1, preferred_element_type=jnp.float32),
            )
            pool_scr[:, hp * Wp:(hp + 1) * Wp] = pooled
        k_eff = lax.dot_general(
            w_th_ref[...], pool_scr[:C8, :],
            (((0,), (0,)), ((), ())),
            preferred_element_type=jnp.float32,
        )
        rr16 = lax.broadcasted_iota(jnp.int32, (PAD, M), 0)
        ones_block = jnp.where(rr16 == 0, 1.0, 0.0).astype(jnp.bfloat16)
        kg_ref[0, :C, :] = k_eff.astype(jnp.bfloat16)
        kg_ref[0, C:C + PAD, :] = ones_block
        kg_ref[0, C + PAD:C + PAD + C2, :] = pool_scr[C8:, :].astype(
            jnp.bfloat16)
        kg_ref[0, C + PAD + C2:, :] = ones_block
        knorm2 = jnp.sum(k_eff * k_eff, axis=0, keepdims=True)
        kmax = jnp.sqrt(jnp.max(knorm2, axis=-1, keepdims=True))
        km_ref[0, :, :] = jnp.broadcast_to(kmax, (8, 128))

    kg, km = pl.pallas_call(
        prep_kernel,
        grid=(B,),
        in_specs=[
            pl.BlockSpec((1, C, N), lambda b: (b, 0, 0)),
            pl.BlockSpec((CP, C), lambda b: (0, 0)),
            pl.BlockSpec((C8, C), lambda b: (0, 0)),
        ],
        out_specs=[
            pl.BlockSpec((1, KG, M), lambda b: (b, 0, 0)),
            pl.BlockSpec((1, 8, 128), lambda b: (b, 0, 0)),
        ],
        out_shape=[
            jax.ShapeDtypeStruct((B, KG, M), jnp.bfloat16),
            jax.ShapeDtypeStruct((B, 8, 128), jnp.float32),
        ],
        scratch_shapes=[
            pltpu.VMEM((CP, N), jnp.float32),
            pltpu.VMEM((CP, M), jnp.float32),
        ],
        compiler_params=pltpu.CompilerParams(
            dimension_semantics=("parallel",),
            vmem_limit_bytes=40 * 1024 * 1024,
        ),
        name="nl_attn_prep",
    )(x_flat, w_pg, w_theta_s)

    def attn_kernel(x_ref, kg_ref, km_ref, w_o_ref, gamma_ref, o_ref):
        xb = x_ref[0]
        k_aug = kg_ref[0, :C + PAD, :]
        g_aug = kg_ref[0, C + PAD:, :]
        kmax = km_ref[0, 0, 0]
        norm2 = jnp.sum(xb * xb, axis=0, keepdims=True)
        bound = jnp.sqrt(norm2) * kmax
        rr16 = lax.broadcasted_iota(jnp.int32, (PAD, QB), 0)
        pad_rows = jnp.where(rr16 == 0, -bound, 0.0).astype(jnp.bfloat16)
        x_aug = jnp.concatenate(
            [xb.astype(jnp.bfloat16), pad_rows], axis=0)
        scores = lax.dot_general(
            x_aug, k_aug, (((0,), (0,)), ((), ())),
            preferred_element_type=jnp.float32,
        )
        e = jnp.exp2(scores.astype(jnp.bfloat16))
        o_aug = lax.dot_general(
            g_aug, e, (((1,), (1,)), ((), ())),
            preferred_element_type=jnp.float32,
        )
        o_midT = o_aug[:C2, :]
        s_row = o_aug[C2:C2 + 1, :]
        oT = jnp.dot(w_o_ref[...], o_midT, preferred_element_type=jnp.float32)
        scale = gamma_ref[0] / s_row
        o_ref[0] = oT * scale + xb

    out_flat = pl.pallas_call(
        attn_kernel,
        grid=(B, N // QB),
        in_specs=[
            pl.BlockSpec((1, C, QB), lambda b, q: (b, 0, q)),
            pl.BlockSpec((1, KG, M), lambda b, q: (b, 0, 0)),
            pl.BlockSpec((1, 8, 128), lambda b, q: (b, 0, 0)),
            pl.BlockSpec((C, C2), lambda b, q: (0, 0)),
            pl.BlockSpec(memory_space=pltpu.SMEM),
        ],
        out_specs=pl.BlockSpec((1, C, QB), lambda b, q: (b, 0, q)),
        out_shape=jax.ShapeDtypeStruct((B, C, N), jnp.float32),
        compiler_params=pltpu.CompilerParams(
            dimension_semantics=("parallel", "arbitrary"),
            vmem_limit_bytes=56 * 1024 * 1024,
        ),
        name="nl_attn_main",
    )(x_flat, kg, km, w_o, gamma_arr)

    return out_flat.reshape(B, C, H, W)


# speedup vs baseline: 1.1610x; 1.0205x over previous
import jax
import jax.numpy as jnp
from jax import lax
from jax.experimental import pallas as pl
from jax.experimental.pallas import tpu as pltpu

_QB = 2048
_LOG2E = 1.4426950408889634


def kernel(x, w_theta, w_phi, w_g, w_o, gamma):
    B, C, H, W = x.shape
    N = H * W
    Hp, Wp = H // 2, W // 2
    M = Hp * Wp
    C8 = w_theta.shape[0]
    C2 = w_g.shape[0]
    CP = C8 + C2
    PAD = 16
    KG = C + PAD + C2 + PAD
    QB = _QB

    x_flat = x.reshape(B, C, N)
    w_pg = jnp.concatenate([w_phi, w_g], axis=0)
    w_theta_s = (w_theta * _LOG2E).astype(jnp.float32)
    gamma_arr = jnp.reshape(gamma, (1,)).astype(jnp.float32)

    def prep_kernel(x_ref, w_pg_ref, w_th_ref, kg_ref, km_ref,
                    full_scr, pool_scr):
        xb = x_ref[0]
        full_scr[...] = jnp.dot(
            w_pg_ref[...], xb, preferred_element_type=jnp.float32
        )
        r = lax.broadcasted_iota(jnp.int32, (W, Wp), 0)
        c = lax.broadcasted_iota(jnp.int32, (W, Wp), 1)
        E0 = jnp.where(r == 2 * c, 1.0, 0.0).astype(jnp.float32)
        E1 = jnp.where(r == 2 * c + 1, 1.0, 0.0).astype(jnp.float32)
        for hp in range(Hp):
            a = full_scr[:, (2 * hp) * W:(2 * hp) * W + W]
            b = full_scr[:, (2 * hp + 1) * W:(2 * hp + 1) * W + W]
            hm = jnp.maximum(a, b)
            pooled = jnp.maximum(
                jnp.dot(hm, E0, preferred_element_type=jnp.float32),
                jnp.dot(hm, E1, preferred_element_type=jnp.float32),
            )
            pool_scr[:, hp * Wp:(hp + 1) * Wp] = pooled
        k_eff = lax.dot_general(
            w_th_ref[...], pool_scr[:C8, :],
            (((0,), (0,)), ((), ())),
            preferred_element_type=jnp.float32,
        )
        rr16 = lax.broadcasted_iota(jnp.int32, (PAD, M), 0)
        ones_block = jnp.where(rr16 == 0, 1.0, 0.0).astype(jnp.bfloat16)
        kg_ref[0, :C, :] = k_eff.astype(jnp.bfloat16)
        kg_ref[0, C:C + PAD, :] = ones_block
        kg_ref[0, C + PAD:C + PAD + C2, :] = pool_scr[C8:, :].astype(
            jnp.bfloat16)
        kg_ref[0, C + PAD + C2:, :] = ones_block
        knorm2 = jnp.sum(k_eff * k_eff, axis=0, keepdims=True)
        kmax = jnp.sqrt(jnp.max(knorm2, axis=-1, keepdims=True))
        km_ref[0, :, :] = jnp.broadcast_to(kmax, (8, 128))

    kg, km = pl.pallas_call(
        prep_kernel,
        grid=(B,),
        in_specs=[
            pl.BlockSpec((1, C, N), lambda b: (b, 0, 0)),
            pl.BlockSpec((CP, C), lambda b: (0, 0)),
            pl.BlockSpec((C8, C), lambda b: (0, 0)),
        ],
        out_specs=[
            pl.BlockSpec((1, KG, M), lambda b: (b, 0, 0)),
            pl.BlockSpec((1, 8, 128), lambda b: (b, 0, 0)),
        ],
        out_shape=[
            jax.ShapeDtypeStruct((B, KG, M), jnp.bfloat16),
            jax.ShapeDtypeStruct((B, 8, 128), jnp.float32),
        ],
        scratch_shapes=[
            pltpu.VMEM((CP, N), jnp.float32),
            pltpu.VMEM((CP, M), jnp.float32),
        ],
        compiler_params=pltpu.CompilerParams(
            dimension_semantics=("parallel",),
            vmem_limit_bytes=40 * 1024 * 1024,
        ),
        name="nl_attn_prep",
    )(x_flat, w_pg, w_theta_s)

    def attn_kernel(x_ref, kg_ref, km_ref, w_o_ref, gamma_ref, o_ref):
        xb = x_ref[0]
        k_aug = kg_ref[0, :C + PAD, :]
        g_aug = kg_ref[0, C + PAD:, :]
        kmax = km_ref[0, 0, 0]
        norm2 = jnp.sum(xb * xb, axis=0, keepdims=True)
        bound = jnp.sqrt(norm2) * kmax
        rr16 = lax.broadcasted_iota(jnp.int32, (PAD, QB), 0)
        pad_rows = jnp.where(rr16 == 0, -bound, 0.0).astype(jnp.bfloat16)
        x_aug = jnp.concatenate(
            [xb.astype(jnp.bfloat16), pad_rows], axis=0)
        KT = 512
        o_aug = None
        for t in range(M // KT):
            k_t = kg_ref[0, :C + PAD, t * KT:(t + 1) * KT]
            g_t = kg_ref[0, C + PAD:, t * KT:(t + 1) * KT]
            scores_t = lax.dot_general(
                x_aug, k_t, (((0,), (0,)), ((), ())),
                preferred_element_type=jnp.float32,
            )
            e_t = jnp.exp2(scores_t.astype(jnp.bfloat16))
            p_t = lax.dot_general(
                g_t, e_t, (((1,), (1,)), ((), ())),
                preferred_element_type=jnp.float32,
            )
            o_aug = p_t if o_aug is None else o_aug + p_t
        o_midT = o_aug[:C2, :]
        s_row = o_aug[C2:C2 + 1, :]
        oT = jnp.dot(w_o_ref[...], o_midT, preferred_element_type=jnp.float32)
        scale = gamma_ref[0] / s_row
        o_ref[0] = oT * scale + xb

    out_flat = pl.pallas_call(
        attn_kernel,
        grid=(B, N // QB),
        in_specs=[
            pl.BlockSpec((1, C, QB), lambda b, q: (b, 0, q)),
            pl.BlockSpec((1, KG, M), lambda b, q: (b, 0, 0)),
            pl.BlockSpec((1, 8, 128), lambda b, q: (b, 0, 0)),
            pl.BlockSpec((C, C2), lambda b, q: (0, 0)),
            pl.BlockSpec(memory_space=pltpu.SMEM),
        ],
        out_specs=pl.BlockSpec((1, C, QB), lambda b, q: (b, 0, q)),
        out_shape=jax.ShapeDtypeStruct((B, C, N), jnp.float32),
        compiler_params=pltpu.CompilerParams(
            dimension_semantics=("parallel", "arbitrary"),
            vmem_limit_bytes=56 * 1024 * 1024,
        ),
        name="nl_attn_main",
    )(x_flat, kg, km, w_o, gamma_arr)

    return out_flat.reshape(B, C, H, W)


# speedup vs baseline: 1.1613x; 1.0002x over previous
import jax
import jax.numpy as jnp
from jax import lax
from jax.experimental import pallas as pl
from jax.experimental.pallas import tpu as pltpu

_QB = 2048
_KT = 512
_LOG2E = 1.4426950408889634


def kernel(x, w_theta, w_phi, w_g, w_o, gamma):
    B, C, H, W = x.shape
    N = H * W
    Hp, Wp = H // 2, W // 2
    M = Hp * Wp
    C8 = w_theta.shape[0]
    C2 = w_g.shape[0]
    CP = C8 + C2
    PAD = 16
    KG = C + PAD + C2 + PAD
    KM_ROW = 8
    QB = _QB

    x_flat = x.reshape(B, C, N)
    gamma_arr = jnp.reshape(gamma, (1,)).astype(jnp.float32)

    def prep_kernel(x_ref, w_th_ref, w_phi_ref, w_g_ref, gamma_ref, kg_ref,
                    full_scr, pool_scr):
        xb = x_ref[0]
        w_pg = jnp.concatenate([w_phi_ref[...], w_g_ref[...]], axis=0)
        full_scr[...] = jnp.dot(
            w_pg, xb, preferred_element_type=jnp.float32
        )
        r = lax.broadcasted_iota(jnp.int32, (W, Wp), 0)
        c = lax.broadcasted_iota(jnp.int32, (W, Wp), 1)
        E0 = jnp.where(r == 2 * c, 1.0, 0.0).astype(jnp.float32)
        E1 = jnp.where(r == 2 * c + 1, 1.0, 0.0).astype(jnp.float32)
        for hp in range(Hp):
            a = full_scr[:, (2 * hp) * W:(2 * hp) * W + W]
            b = full_scr[:, (2 * hp + 1) * W:(2 * hp + 1) * W + W]
            hm = jnp.maximum(a, b)
            pooled = jnp.maximum(
                jnp.dot(hm, E0, preferred_element_type=jnp.float32),
                jnp.dot(hm, E1, preferred_element_type=jnp.float32),
            )
            pool_scr[:, hp * Wp:(hp + 1) * Wp] = pooled
        k_eff = lax.dot_general(
            w_th_ref[...] * _LOG2E, pool_scr[:C8, :],
            (((0,), (0,)), ((), ())),
            preferred_element_type=jnp.float32,
        )
        rr16 = lax.broadcasted_iota(jnp.int32, (PAD, M), 0)
        ones_block = jnp.where(rr16 == 0, 1.0, 0.0).astype(jnp.bfloat16)
        kg_ref[0, :C, :] = k_eff.astype(jnp.bfloat16)
        kg_ref[0, C:C + PAD, :] = ones_block
        kg_ref[0, C + PAD:C + PAD + C2, :] = (
            pool_scr[C8:, :] * gamma_ref[0]).astype(jnp.bfloat16)
        knorm2 = jnp.sum(k_eff * k_eff, axis=0, keepdims=True)
        kmax = jnp.sqrt(jnp.max(knorm2, axis=-1, keepdims=True))
        tail = jnp.where(
            rr16 == 0, 1.0, jnp.where(rr16 == KM_ROW, kmax, 0.0))
        kg_ref[0, C + PAD + C2:, :] = tail.astype(jnp.bfloat16)

    kg = pl.pallas_call(
        prep_kernel,
        grid=(B,),
        in_specs=[
            pl.BlockSpec((1, C, N), lambda b: (b, 0, 0)),
            pl.BlockSpec((C8, C), lambda b: (0, 0)),
            pl.BlockSpec((C8, C), lambda b: (0, 0)),
            pl.BlockSpec((C2, C), lambda b: (0, 0)),
            pl.BlockSpec(memory_space=pltpu.SMEM),
        ],
        out_specs=pl.BlockSpec((1, KG, M), lambda b: (b, 0, 0)),
        out_shape=jax.ShapeDtypeStruct((B, KG, M), jnp.bfloat16),
        scratch_shapes=[
            pltpu.VMEM((CP, N), jnp.float32),
            pltpu.VMEM((CP, M), jnp.float32),
        ],
        compiler_params=pltpu.CompilerParams(
            dimension_semantics=("parallel",),
            vmem_limit_bytes=40 * 1024 * 1024,
        ),
        name="nl_attn_prep",
    )(x_flat, w_theta, w_phi, w_g, gamma_arr)

    def attn_kernel(x_ref, kg_ref, w_o_ref, o_ref):
        xb = x_ref[0]
        kmax = kg_ref[0, C + PAD + C2 + KM_ROW:C + PAD + C2 + KM_ROW + 1, 0:1]
        norm2 = jnp.sum(xb * xb, axis=0, keepdims=True)
        bound = jnp.sqrt(norm2) * kmax.astype(jnp.float32)
        rr16 = lax.broadcasted_iota(jnp.int32, (PAD, QB), 0)
        pad_rows = jnp.where(rr16 == 0, -bound, 0.0).astype(jnp.bfloat16)
        x_aug = jnp.concatenate(
            [xb.astype(jnp.bfloat16), pad_rows], axis=0)
        KT = _KT
        o_aug = None
        for t in range(M // KT):
            k_t = kg_ref[0, :C + PAD, t * KT:(t + 1) * KT]
            g_t = kg_ref[0, C + PAD:, t * KT:(t + 1) * KT]
            scores_t = lax.dot_general(
                x_aug, k_t, (((0,), (0,)), ((), ())),
                preferred_element_type=jnp.float32,
            )
            e_t = jnp.exp2(scores_t.astype(jnp.bfloat16))
            p_t = lax.dot_general(
                g_t, e_t, (((1,), (1,)), ((), ())),
                preferred_element_type=jnp.float32,
            )
            o_aug = p_t if o_aug is None else o_aug + p_t
        o_midT = o_aug[:C2, :]
        s_row = o_aug[C2:C2 + 1, :]
        oT = jnp.dot(w_o_ref[...], o_midT, preferred_element_type=jnp.float32)
        o_ref[0] = oT / s_row + xb

    out_flat = pl.pallas_call(
        attn_kernel,
        grid=(B, N // QB),
        in_specs=[
            pl.BlockSpec((1, C, QB), lambda b, q: (b, 0, q)),
            pl.BlockSpec((1, KG, M), lambda b, q: (b, 0, 0)),
            pl.BlockSpec((C, C2), lambda b, q: (0, 0)),
        ],
        out_specs=pl.BlockSpec((1, C, QB), lambda b, q: (b, 0, q)),
        out_shape=jax.ShapeDtypeStruct((B, C, N), jnp.float32),
        compiler_params=pltpu.CompilerParams(
            dimension_semantics=("parallel", "arbitrary"),
            vmem_limit_bytes=56 * 1024 * 1024,
        ),
        name="nl_attn_main",
    )(x_flat, kg, w_o)

    return out_flat.reshape(B, C, H, W)


# speedup vs baseline: 1.1705x; 1.0080x over previous
import jax
import jax.numpy as jnp
from jax import lax
from jax.experimental import pallas as pl
from jax.experimental.pallas import tpu as pltpu

_QB = 2048
_KT = 512
_LOG2E = 1.4426950408889634


def kernel(x, w_theta, w_phi, w_g, w_o, gamma):
    B, C, H, W = x.shape
    N = H * W
    Hp, Wp = H // 2, W // 2
    M = Hp * Wp
    C8 = w_theta.shape[0]
    C2 = w_g.shape[0]
    CP = C8 + C2
    PAD = 16
    KG = C + PAD + C2 + PAD
    QB = _QB

    x_flat = x.reshape(B, C, N)
    gamma_arr = jnp.reshape(gamma, (1,)).astype(jnp.float32)

    def fused_kernel(x_full_ref, x_ref, w_th_ref, w_phi_ref, w_g_ref,
                     w_o_ref, gamma_ref, o_ref,
                     kg_scr, km_scr, full_scr, pool_scr):
        q = pl.program_id(1)

        @pl.when(q == 0)
        def _prep():
            xb = x_full_ref[0]
            w_pg = jnp.concatenate([w_phi_ref[...], w_g_ref[...]], axis=0)
            full_scr[...] = jnp.dot(
                w_pg, xb, preferred_element_type=jnp.float32
            )
            r = lax.broadcasted_iota(jnp.int32, (W, Wp), 0)
            c = lax.broadcasted_iota(jnp.int32, (W, Wp), 1)
            E0 = jnp.where(r == 2 * c, 1.0, 0.0).astype(jnp.float32)
            E1 = jnp.where(r == 2 * c + 1, 1.0, 0.0).astype(jnp.float32)
            for hp in range(Hp):
                a = full_scr[:, (2 * hp) * W:(2 * hp) * W + W]
                b = full_scr[:, (2 * hp + 1) * W:(2 * hp + 1) * W + W]
                hm = jnp.maximum(a, b)
                pooled = jnp.maximum(
                    jnp.dot(hm, E0, preferred_element_type=jnp.float32),
                    jnp.dot(hm, E1, preferred_element_type=jnp.float32),
                )
                pool_scr[:, hp * Wp:(hp + 1) * Wp] = pooled
            k_eff = lax.dot_general(
                w_th_ref[...] * _LOG2E, pool_scr[:C8, :],
                (((0,), (0,)), ((), ())),
                preferred_element_type=jnp.float32,
            )
            rr16 = lax.broadcasted_iota(jnp.int32, (PAD, M), 0)
            ones_block = jnp.where(rr16 == 0, 1.0, 0.0).astype(jnp.bfloat16)
            kg_scr[:C, :] = k_eff.astype(jnp.bfloat16)
            kg_scr[C:C + PAD, :] = ones_block
            kg_scr[C + PAD:C + PAD + C2, :] = (
                pool_scr[C8:, :] * gamma_ref[0]).astype(jnp.bfloat16)
            kg_scr[C + PAD + C2:, :] = ones_block
            knorm2 = jnp.sum(k_eff * k_eff, axis=0, keepdims=True)
            km_scr[...] = jnp.sqrt(
                jnp.max(knorm2, axis=-1, keepdims=True))

        xb = x_ref[0]
        norm2 = jnp.sum(xb * xb, axis=0, keepdims=True)
        bound = jnp.sqrt(norm2) * km_scr[...]
        rr = lax.broadcasted_iota(jnp.int32, (PAD, QB), 0)
        pad_rows = jnp.where(rr == 0, -bound, 0.0).astype(jnp.bfloat16)
        x_aug = jnp.concatenate(
            [xb.astype(jnp.bfloat16), pad_rows], axis=0)
        o_aug = None
        for t in range(M // _KT):
            k_t = kg_scr[:C + PAD, t * _KT:(t + 1) * _KT]
            g_t = kg_scr[C + PAD:, t * _KT:(t + 1) * _KT]
            scores_t = lax.dot_general(
                x_aug, k_t, (((0,), (0,)), ((), ())),
                preferred_element_type=jnp.float32,
            )
            e_t = jnp.exp2(scores_t.astype(jnp.bfloat16))
            p_t = lax.dot_general(
                g_t, e_t, (((1,), (1,)), ((), ())),
                preferred_element_type=jnp.float32,
            )
            o_aug = p_t if o_aug is None else o_aug + p_t
        o_midT = o_aug[:C2, :]
        s_row = o_aug[C2:C2 + 1, :]
        oT = jnp.dot(w_o_ref[...], o_midT, preferred_element_type=jnp.float32)
        o_ref[0] = oT / s_row + xb

    out_flat = pl.pallas_call(
        fused_kernel,
        grid=(B, N // QB),
        in_specs=[
            pl.BlockSpec((1, C, N), lambda b, q: (b, 0, 0)),
            pl.BlockSpec((1, C, QB), lambda b, q: (b, 0, q)),
            pl.BlockSpec((C8, C), lambda b, q: (0, 0)),
            pl.BlockSpec((C8, C), lambda b, q: (0, 0)),
            pl.BlockSpec((C2, C), lambda b, q: (0, 0)),
            pl.BlockSpec((C, C2), lambda b, q: (0, 0)),
            pl.BlockSpec(memory_space=pltpu.SMEM),
        ],
        out_specs=pl.BlockSpec((1, C, QB), lambda b, q: (b, 0, q)),
        out_shape=jax.ShapeDtypeStruct((B, C, N), jnp.float32),
        scratch_shapes=[
            pltpu.VMEM((KG, M), jnp.bfloat16),
            pltpu.VMEM((1, 1), jnp.float32),
            pltpu.VMEM((CP, N), jnp.float32),
            pltpu.VMEM((CP, M), jnp.float32),
        ],
        compiler_params=pltpu.CompilerParams(
            dimension_semantics=("parallel", "arbitrary"),
            vmem_limit_bytes=56 * 1024 * 1024,
        ),
        name="nl_attn_fused",
    )(x_flat, x_flat, w_theta, w_phi, w_g, w_o, gamma_arr)

    return out_flat.reshape(B, C, H, W)


# speedup vs baseline: 1.2015x; 1.0264x over previous
import jax
import jax.numpy as jnp
from jax import lax
from jax.experimental import pallas as pl
from jax.experimental.pallas import tpu as pltpu

_QB = 2048
_KT = 512
_LOG2E = 1.4426950408889634


def kernel(x, w_theta, w_phi, w_g, w_o, gamma):
    B, C, H, W = x.shape
    N = H * W
    Hp, Wp = H // 2, W // 2
    M = Hp * Wp
    C8 = w_theta.shape[0]
    C2 = w_g.shape[0]
    CP = C8 + C2
    PAD = 16
    KG = C + PAD + C2 + PAD
    QB = _QB

    x_flat = x.reshape(B, C, N)
    gamma_arr = jnp.reshape(gamma, (1,)).astype(jnp.float32)

    def fused_kernel(x_full_ref, x_ref, w_th_ref, w_phi_ref, w_g_ref,
                     w_o_ref, gamma_ref, o_ref,
                     kg_scr, km_scr, full_scr, pool_scr):
        q = pl.program_id(1)

        @pl.when(q == 0)
        def _prep():
            xb = x_full_ref[0]
            w_pg = jnp.concatenate([w_phi_ref[...], w_g_ref[...]], axis=0)
            full_scr[...] = jnp.dot(
                w_pg, xb, preferred_element_type=jnp.float32
            )
            r = lax.broadcasted_iota(jnp.int32, (W, Wp), 0)
            c = lax.broadcasted_iota(jnp.int32, (W, Wp), 1)
            E0 = jnp.where(r == 2 * c, 1.0, 0.0).astype(jnp.float32)
            E1 = jnp.where(r == 2 * c + 1, 1.0, 0.0).astype(jnp.float32)
            for hp in range(Hp):
                a = full_scr[:, (2 * hp) * W:(2 * hp) * W + W]
                b = full_scr[:, (2 * hp + 1) * W:(2 * hp + 1) * W + W]
                hm = jnp.maximum(a, b)
                pooled = jnp.maximum(
                    jnp.dot(hm, E0, preferred_element_type=jnp.float32),
                    jnp.dot(hm, E1, preferred_element_type=jnp.float32),
                )
                pool_scr[:, hp * Wp:(hp + 1) * Wp] = pooled
            k_eff = lax.dot_general(
                w_th_ref[...] * _LOG2E, pool_scr[:C8, :],
                (((0,), (0,)), ((), ())),
                preferred_element_type=jnp.float32,
            )
            rr16 = lax.broadcasted_iota(jnp.int32, (PAD, M), 0)
            ones_block = jnp.where(rr16 == 0, 1.0, 0.0).astype(jnp.bfloat16)
            kg_scr[:C, :] = k_eff.astype(jnp.bfloat16)
            kg_scr[C:C + PAD, :] = ones_block
            kg_scr[C + PAD:C + PAD + C2, :] = (
                pool_scr[C8:, :] * gamma_ref[0]).astype(jnp.bfloat16)
            kg_scr[C + PAD + C2:, :] = ones_block
            knorm2 = jnp.sum(k_eff * k_eff, axis=0, keepdims=True)
            km_scr[...] = jnp.sqrt(
                jnp.max(knorm2, axis=-1, keepdims=True))

        xb = x_ref[0]
        norm2 = jnp.sum(xb * xb, axis=0, keepdims=True)
        bound = jnp.sqrt(norm2) * km_scr[...]
        rr = lax.broadcasted_iota(jnp.int32, (PAD, QB), 0)
        pad_rows = jnp.where(rr == 0, -bound, 0.0).astype(jnp.bfloat16)
        x_aug = jnp.concatenate(
            [xb.astype(jnp.bfloat16), pad_rows], axis=0)
        QS = QB // 2
        xs = [x_aug[:, s * QS:(s + 1) * QS] for s in range(2)]
        acc = [None, None]
        for t in range(M // _KT):
            k_t = kg_scr[:C + PAD, t * _KT:(t + 1) * _KT]
            g_t = kg_scr[C + PAD:, t * _KT:(t + 1) * _KT]
            for s in range(2):
                scores_t = lax.dot_general(
                    xs[s], k_t, (((0,), (0,)), ((), ())),
                    preferred_element_type=jnp.float32,
                )
                e_t = jnp.exp2(scores_t.astype(jnp.bfloat16))
                p_t = lax.dot_general(
                    g_t, e_t, (((1,), (1,)), ((), ())),
                    preferred_element_type=jnp.float32,
                )
                acc[s] = p_t if acc[s] is None else acc[s] + p_t
        for s in range(2):
            o_aug = acc[s]
            o_midT = o_aug[:C2, :]
            s_row = o_aug[C2:C2 + 1, :]
            oT = jnp.dot(
                w_o_ref[...], o_midT, preferred_element_type=jnp.float32)
            o_ref[0, :, s * QS:(s + 1) * QS] = (
                oT / s_row + xb[:, s * QS:(s + 1) * QS])

    out_flat = pl.pallas_call(
        fused_kernel,
        grid=(B, N // QB),
        in_specs=[
            pl.BlockSpec((1, C, N), lambda b, q: (b, 0, 0)),
            pl.BlockSpec((1, C, QB), lambda b, q: (b, 0, q)),
            pl.BlockSpec((C8, C), lambda b, q: (0, 0)),
            pl.BlockSpec((C8, C), lambda b, q: (0, 0)),
            pl.BlockSpec((C2, C), lambda b, q: (0, 0)),
            pl.BlockSpec((C, C2), lambda b, q: (0, 0)),
            pl.BlockSpec(memory_space=pltpu.SMEM),
        ],
        out_specs=pl.BlockSpec((1, C, QB), lambda b, q: (b, 0, q)),
        out_shape=jax.ShapeDtypeStruct((B, C, N), jnp.float32),
        scratch_shapes=[
            pltpu.VMEM((KG, M), jnp.bfloat16),
            pltpu.VMEM((1, 1), jnp.float32),
            pltpu.VMEM((CP, N), jnp.float32),
            pltpu.VMEM((CP, M), jnp.float32),
        ],
        compiler_params=pltpu.CompilerParams(
            dimension_semantics=("parallel", "arbitrary"),
            vmem_limit_bytes=56 * 1024 * 1024,
        ),
        name="nl_attn_fused",
    )(x_flat, x_flat, w_theta, w_phi, w_g, w_o, gamma_arr)

    return out_flat.reshape(B, C, H, W)
